# Initial kernel scaffold; baseline (speedup 1.0000x reference)
import jax
import jax.numpy as jnp
from jax.experimental import pallas as pl


def kernel(bbox20, p20, c20, bbox40, p40, c40, bbox80, p80, c80, training):
    raise NotImplementedError("write your pallas kernel here")


# speedup vs baseline: 7.4552x; 7.4552x over previous
import functools

import jax
import jax.numpy as jnp
from jax.experimental import pallas as pl
from jax.experimental.pallas import tpu as pltpu

NUM_CLASSES = 80
IOU_T = 0.5
SCORE_T = 0.25
MAX_B = 100
NEG = -1e30

N_TOT = 25200
N_PAD = 25216


def _score_kernel(c_ref, p_ref, o_ref):
    x = c_ref[...]
    p = p_ref[...]
    m = jnp.max(x, axis=1, keepdims=True)
    iota = jax.lax.broadcasted_iota(jnp.int32, x.shape, 1)
    idx = jnp.min(jnp.where(x == m, iota, NUM_CLASSES), axis=1, keepdims=True)
    s = p * idx.astype(jnp.float32)
    o_ref[...] = jnp.where(s > SCORE_T, s, NEG)


def _scores(c, p, rows=1200):
    m = c.shape[0]
    return pl.pallas_call(
        _score_kernel,
        grid=(m // rows,),
        in_specs=[
            pl.BlockSpec((rows, NUM_CLASSES), lambda i: (i, 0)),
            pl.BlockSpec((rows, 1), lambda i: (i, 0)),
        ],
        out_specs=pl.BlockSpec((rows, 1), lambda i: (i, 0)),
        out_shape=jax.ShapeDtypeStruct((m, 1), jnp.float32),
    )(c, p)


def _nms_kernel(s_ref, y1_ref, x1_ref, y2_ref, x2_ref,
                oy1_ref, ox1_ref, oy2_ref, ox2_ref, os_ref, cnt_ref,
                s_scr):
    s_scr[...] = s_ref[...]
    y1 = y1_ref[...]
    x1 = x1_ref[...]
    y2 = y2_ref[...]
    x2 = x2_ref[...]
    areas = jnp.maximum(y2 - y1, 0.0) * jnp.maximum(x2 - x1, 0.0)
    lane = jax.lax.broadcasted_iota(jnp.int32, s_scr.shape, 1)

    def step(t, carry):
        s = s_scr[...]
        m = jnp.max(s, axis=1, keepdims=True)
        idx = jnp.min(jnp.where(s == m, lane, N_PAD), axis=1, keepdims=True)
        onehot = lane == idx
        valid = m > NEG
        by1 = jnp.max(jnp.where(onehot, y1, NEG), axis=1, keepdims=True)
        bx1 = jnp.max(jnp.where(onehot, x1, NEG), axis=1, keepdims=True)
        by2 = jnp.max(jnp.where(onehot, y2, NEG), axis=1, keepdims=True)
        bx2 = jnp.max(jnp.where(onehot, x2, NEG), axis=1, keepdims=True)
        ba = jnp.maximum(by2 - by1, 0.0) * jnp.maximum(bx2 - bx1, 0.0)
        yy1 = jnp.maximum(by1, y1)
        xx1 = jnp.maximum(bx1, x1)
        yy2 = jnp.minimum(by2, y2)
        xx2 = jnp.minimum(bx2, x2)
        inter = jnp.maximum(yy2 - yy1, 0.0) * jnp.maximum(xx2 - xx1, 0.0)
        denom = jnp.maximum(ba + areas - inter, 1e-9)
        kill = (inter > IOU_T * denom) | onehot
        s_scr[...] = jnp.where(valid, jnp.where(kill, NEG, s), s)
        vf = valid.astype(jnp.float32)
        oy1_ref[pl.ds(t, 1), :] = (vf * jnp.clip(by1, 0.0, 1.0)).T
        ox1_ref[pl.ds(t, 1), :] = (vf * jnp.clip(bx1, 0.0, 1.0)).T
        oy2_ref[pl.ds(t, 1), :] = (vf * jnp.clip(by2, 0.0, 1.0)).T
        ox2_ref[pl.ds(t, 1), :] = (vf * jnp.clip(bx2, 0.0, 1.0)).T
        os_ref[pl.ds(t, 1), :] = (vf * m).T
        return carry

    jax.lax.fori_loop(0, MAX_B, step, 0, unroll=False)
    cnt_ref[...] = jnp.sum((os_ref[...] > 0.0).astype(jnp.int32), axis=0,
                           keepdims=True)


def _nms(s, y1, x1, y2, x2):
    b = s.shape[0]
    outs = pl.pallas_call(
        _nms_kernel,
        out_shape=[jax.ShapeDtypeStruct((MAX_B, b), jnp.float32)] * 5
        + [jax.ShapeDtypeStruct((1, b), jnp.int32)],
        scratch_shapes=[pltpu.VMEM((b, N_PAD), jnp.float32)],
    )(s, y1, x1, y2, x2)
    return outs


def kernel(bbox20, p20, c20, bbox40, p40, c40, bbox80, p80, c80,
           training=False):
    B = bbox20.shape[0]
    s20 = _scores(c20.reshape(-1, NUM_CLASSES), p20.reshape(-1, 1))
    s40 = _scores(c40.reshape(-1, NUM_CLASSES), p40.reshape(-1, 1))
    s80 = _scores(c80.reshape(-1, NUM_CLASSES), p80.reshape(-1, 1))
    s = jnp.concatenate(
        [s20.reshape(B, -1), s40.reshape(B, -1), s80.reshape(B, -1)], axis=1)
    bx = jnp.concatenate(
        [bbox20.reshape(B, -1, 4), bbox40.reshape(B, -1, 4),
         bbox80.reshape(B, -1, 4)], axis=1)
    pad = N_PAD - N_TOT
    s = jnp.pad(s, ((0, 0), (0, pad)), constant_values=NEG)
    y1 = jnp.pad(bx[:, :, 0], ((0, 0), (0, pad)))
    x1 = jnp.pad(bx[:, :, 1], ((0, 0), (0, pad)))
    y2 = jnp.pad(bx[:, :, 2], ((0, 0), (0, pad)))
    x2 = jnp.pad(bx[:, :, 3], ((0, 0), (0, pad)))
    oy1, ox1, oy2, ox2, os, cnt = _nms(s, y1, x1, y2, x2)
    pred = jnp.stack(
        [oy1.T, ox1.T, oy2.T, ox2.T, os.T, jnp.zeros_like(os.T)], axis=-1)
    return pred, cnt.reshape(B)
